# baseline (speedup 1.0000x reference)
import functools

import jax
import jax.numpy as jnp
from jax import lax
from jax.experimental import pallas as pl
from jax.experimental.pallas import tpu as pltpu
from jax.experimental.pallas import tpu_sc as plsc

VOCAB = 1_000_000
D = 64
B_TOKENS = 4096
SEQ = 200
B = B_TOKENS * SEQ

NC = 2
NS = 16
NW = NC * NS
B_PER_W = B // NW
BLK = 128
NBLK = B_PER_W // BLK


def _body(idx_hbm, table_hbm, out_hbm, idx_v, rows_v, g0, g1, o0, o1):
    wid = lax.axis_index("s") * NC + lax.axis_index("c")
    base = wid * B_PER_W

    pltpu.sync_copy(idx_hbm.at[wid], idx_v)

    gsems = (g0, g1)
    osems = (o0, o1)

    def gather(j, b):
        return pltpu.make_async_copy(
            table_hbm.at[idx_v.at[j]], rows_v.at[b], gsems[b])

    def writeback(j, b):
        return pltpu.make_async_copy(
            rows_v.at[b], out_hbm.at[pl.ds(base + j * BLK, BLK)], osems[b])

    gather(0, 0).start()

    @pl.loop(0, NBLK // 2)
    def _(s):
        for b in range(2):
            j = 2 * s + b
            gather(j, b).wait()
            writeback(j, b).start()

            @pl.when(j >= 1)
            def _():
                writeback(j - 1, 1 - b).wait()

            @pl.when(j + 1 < NBLK)
            def _():
                gather(j + 1, 1 - b).start()

    writeback(NBLK - 1, 1).wait()


@functools.cache
def _build():
    return pl.kernel(
        _body,
        out_type=jax.ShapeDtypeStruct((B, D), jnp.float32),
        mesh=plsc.VectorSubcoreMesh(
            core_axis_name="c", subcore_axis_name="s",
            num_cores=NC, num_subcores=NS),
        compiler_params=pltpu.CompilerParams(use_tc_tiling_on_sc=False),
        scratch_types=[
            pltpu.VMEM((NBLK, BLK), jnp.int32),
            pltpu.VMEM((2, BLK, D), jnp.float32),
            pltpu.SemaphoreType.DMA,
            pltpu.SemaphoreType.DMA,
            pltpu.SemaphoreType.DMA,
            pltpu.SemaphoreType.DMA,
        ],
    )


def kernel(token_ids, weight):
    idx = token_ids.reshape(NW, NBLK, BLK).astype(jnp.int32)
    out = _build()(idx, weight)
    return out.reshape(B_TOKENS, SEQ, D)


# speedup vs baseline: 1.0775x; 1.0775x over previous
import functools

import jax
import jax.numpy as jnp
from jax import lax
from jax.experimental import pallas as pl
from jax.experimental.pallas import tpu as pltpu
from jax.experimental.pallas import tpu_sc as plsc

VOCAB = 1_000_000
D = 64
B_TOKENS = 4096
SEQ = 200
B = B_TOKENS * SEQ

NC = 2
NS = 16
NW = NC * NS
B_PER_W = B // NW
BLK = 128
NBLK = B_PER_W // BLK


NBUF = 8
AHEAD = 4


def _body(idx_hbm, table_hbm, out_hbm, idx_v, rows_v, *sems):
    wid = lax.axis_index("s") * NC + lax.axis_index("c")
    base = wid * B_PER_W

    pltpu.sync_copy(idx_hbm.at[wid], idx_v)

    gsems = sems[:NBUF]
    osems = sems[NBUF:]

    def gather(j, b):
        return pltpu.make_async_copy(
            table_hbm.at[idx_v.at[j]], rows_v.at[b], gsems[b])

    def writeback(j, b):
        return pltpu.make_async_copy(
            rows_v.at[b], out_hbm.at[pl.ds(base + j * BLK, BLK)], osems[b])

    for g in range(AHEAD):
        gather(g, g).start()

    @pl.loop(0, NBLK // NBUF)
    def _(s):
        for b in range(NBUF):
            j = s * NBUF + b
            gather(j, b).wait()
            writeback(j, b).start()

            @pl.when(j >= AHEAD)
            def _():
                writeback(j - AHEAD, (b - AHEAD) % NBUF).wait()

            @pl.when(j + AHEAD < NBLK)
            def _():
                gather(j + AHEAD, (b + AHEAD) % NBUF).start()

    for j in range(NBLK - AHEAD, NBLK):
        writeback(j, j % NBUF).wait()


@functools.cache
def _build():
    return pl.kernel(
        _body,
        out_type=jax.ShapeDtypeStruct((B, D), jnp.float32),
        mesh=plsc.VectorSubcoreMesh(
            core_axis_name="c", subcore_axis_name="s",
            num_cores=NC, num_subcores=NS),
        compiler_params=pltpu.CompilerParams(use_tc_tiling_on_sc=False),
        scratch_types=[
            pltpu.VMEM((NBLK, BLK), jnp.int32),
            pltpu.VMEM((NBUF, BLK, D), jnp.float32),
        ] + [pltpu.SemaphoreType.DMA] * (2 * NBUF),
    )


def kernel(token_ids, weight):
    idx = token_ids.reshape(NW, NBLK, BLK).astype(jnp.int32)
    out = _build()(idx, weight)
    return out.reshape(B_TOKENS, SEQ, D)
